# baseline (speedup 1.0000x reference)
import dataclasses
import functools

import jax
import jax.numpy as jnp
from jax import lax
from jax.experimental import pallas as pl
from jax.experimental.pallas import tpu as pltpu
from jax.experimental.pallas import tpu_sc as plsc

N = 10000
C = 128
K = 25
KP = 32

NW = 32
AD = 4
RG = 4
GD = 2
NG = 39
ROWS_PER = 320
NGRP = ROWS_PER // RG
NP = NW * ROWS_PER
XPAD_ROWS = N + 8


def _sage_body(a_hbm, xpad_hbm, out_hbm, abuf0, abuf1, abuf2, abuf3,
               idxb, gbuf, aggb, clist, sa0, sa1, sa2, sa3, sg0, sg1):
    wid = lax.axis_index("s") * 2 + lax.axis_index("c")
    base = wid * ROWS_PER
    iota16 = lax.iota(jnp.int32, 16)
    nfill = jnp.full((16,), N, jnp.int32)
    zeros16 = jnp.zeros((16,), jnp.int32)
    sa = (sa0, sa1, sa2, sa3)
    sg = (sg0, sg1)
    abufs = (abuf0, abuf1, abuf2, abuf3)
    lane_eq = [iota16 == t for t in range(16)]

    def a_row(r):
        return jnp.minimum(base + r, N - 1)

    def start_a(r, p):
        pltpu.make_async_copy(a_hbm.at[pl.ds(a_row(r) * N, N)], abufs[p],
                              sa[p]).start()

    def wait_a(p):
        pltpu.make_async_copy(a_hbm.at[pl.ds(0, N)], abufs[p],
                              sa[p]).wait()

    def start_g(pg):
        pltpu.make_async_copy(xpad_hbm.at[idxb.at[pg]], gbuf.at[pg],
                              sg[pg]).start()

    def wait_g(pg):
        pltpu.make_async_copy(xpad_hbm.at[idxb.at[pg]], gbuf.at[pg],
                              sg[pg]).wait()

    def scan_row(pa, pg, rg):
        idxb[pg, pl.ds(rg * KP, 16)] = nfill
        idxb[pg, pl.ds(rg * KP + 16, 16)] = nfill

        def compact_chunks(flags, g, gcnt):
            m2 = flags != 0
            pos = gcnt + plsc.cumsum(m2.astype(jnp.int32)) - 1
            sm = jnp.logical_and(m2, pos < KP)
            posc = jnp.minimum(pos, KP - 1)
            plsc.store_scatter(clist, [posc], iota16 + g * 16, mask=sm)
            return gcnt + plsc.all_reduce_population_count(m2)

        def group(g, gcnt):
            flags = zeros16
            for t in range(16):
                v = abufs[pa][pl.ds(g * 256 + t * 16, 16)]
                nz = lax.shift_left(plsc.bitcast(v, jnp.int32), 1) != 0
                pc = plsc.all_reduce_population_count(nz)
                flags = jnp.where(lane_eq[t], pc, flags)
            return compact_chunks(flags, g, gcnt)

        gcnt = lax.fori_loop(0, NG, group, zeros16, unroll=False)
        vlast = abufs[pa][pl.ds(NG * 256, 16)]
        nzl = lax.shift_left(plsc.bitcast(vlast, jnp.int32), 1) != 0
        pcl = plsc.all_reduce_population_count(nzl)
        gcnt = compact_chunks(jnp.where(lane_eq[0], pcl, zeros16), NG, gcnt)
        nchunks = jnp.minimum(jnp.max(gcnt), KP)

        def chunk(j, cnt):
            jv = jnp.full((16,), 0, jnp.int32) + j
            cid = plsc.load_gather(clist, [jv])
            cols = cid * 16 + iota16
            v = plsc.load_gather(abufs[pa], [cols])
            m = lax.shift_left(plsc.bitcast(v, jnp.int32), 1) != 0
            pos = cnt + plsc.cumsum(m.astype(jnp.int32)) - 1
            sm = jnp.logical_and(m, pos < K)
            posc = jnp.minimum(pos, KP - 1) + rg * KP
            plsc.store_scatter(idxb.at[pg], [posc], cols, mask=sm)
            return cnt + plsc.all_reduce_population_count(m)

        lax.fori_loop(0, nchunks, chunk, zeros16, unroll=False)

        sl0 = jnp.full((16,), rg * KP, jnp.int32)
        idx0 = plsc.load_gather(idxb.at[pg], [sl0])
        hi = idxb[pg, pl.ds(rg * KP + 16, 16)]
        idxb[pg, pl.ds(rg * KP + 16, 16)] = jnp.where(iota16 >= K - 16,
                                                      idx0, hi)

    def max_row(pg, rg, rmax):
        def mstep(k, accs):
            return tuple(
                jnp.maximum(a, gbuf[pg, rg * KP + k, pl.ds(cch * 16, 16)])
                for cch, a in enumerate(accs))
        accs = tuple(gbuf[pg, rg * KP, pl.ds(cch * 16, 16)]
                     for cch in range(C // 16))
        accs = lax.fori_loop(1, KP, mstep, accs, unroll=4)
        for cch in range(C // 16):
            aggb[rmax, pl.ds(cch * 16, 16)] = accs[cch]

    for u in range(AD):
        start_a(u, u)

    @pl.loop(0, NGRP + GD, step=GD)
    def _(g0):
        for ug in range(GD):
            grp = g0 + ug

            @pl.when(grp < NGRP)
            def _():
                for rg in range(RG):
                    row = grp * RG + rg
                    wait_a(rg)
                    scan_row(rg, ug, rg)

                    @pl.when(row + AD < ROWS_PER)
                    def _():
                        start_a(row + AD, rg)
                start_g(ug)

            gmax = grp - 1

            @pl.when(jnp.logical_and(gmax >= 0, gmax < NGRP))
            def _():
                wait_g(1 - ug)
                for rg in range(RG):
                    max_row(1 - ug, rg, gmax * RG + rg)

    pltpu.sync_copy(aggb, out_hbm.at[pl.ds(base, ROWS_PER)])


def _sage_sc(A1, xpad):
    mesh = plsc.VectorSubcoreMesh(core_axis_name="c", subcore_axis_name="s")
    cp = pltpu.CompilerParams()
    if "needs_layout_passes" in pltpu.CompilerParams.__dataclass_fields__:
        cp = dataclasses.replace(cp, needs_layout_passes=False)
    kfn = functools.partial(
        pl.kernel,
        mesh=mesh,
        compiler_params=cp,
        out_type=jax.ShapeDtypeStruct((NP, C), jnp.float32),
        scratch_types=(
            [pltpu.VMEM((N,), jnp.float32)] * AD
            + [pltpu.VMEM((GD, RG * KP), jnp.int32),
               pltpu.VMEM((GD, RG * KP, C), jnp.float32),
               pltpu.VMEM((ROWS_PER, C), jnp.float32),
               pltpu.VMEM((KP,), jnp.int32)]
            + [pltpu.SemaphoreType.DMA] * (AD + GD)
        ),
    )(_sage_body)
    return kfn(A1, xpad)


def _mlp_body(x_ref, a_ref, w1_ref, w2_ref, b_ref, o_ref):
    acc = jnp.dot(x_ref[...], w1_ref[...], preferred_element_type=jnp.float32)
    acc += jnp.dot(a_ref[...], w2_ref[...], preferred_element_type=jnp.float32)
    o_ref[...] = jnp.maximum(acc + b_ref[...], 0.0)


def _mlp(X2, agg, W, b):
    MB = 1000
    return pl.pallas_call(
        _mlp_body,
        grid=(N // MB,),
        in_specs=[
            pl.BlockSpec((MB, C), lambda i: (i, 0)),
            pl.BlockSpec((MB, C), lambda i: (i, 0)),
            pl.BlockSpec((C, C), lambda i: (0, 0)),
            pl.BlockSpec((C, C), lambda i: (0, 0)),
            pl.BlockSpec((1, C), lambda i: (0, 0)),
        ],
        out_specs=pl.BlockSpec((MB, C), lambda i: (i, 0)),
        out_shape=jax.ShapeDtypeStruct((N, C), jnp.float32),
    )(X2, agg, W[:C], W[C:], b[None])


def kernel(A, X, agg_weights, agg_bias):
    X2 = X[0]
    A1 = jnp.reshape(A, (N * N,))
    xpad = jnp.pad(X2, ((0, XPAD_ROWS - N), (0, 0)))
    agg = _sage_sc(A1, xpad)[:N]
    out = _mlp(X2, agg, agg_weights, agg_bias)
    return out[None]


# speedup vs baseline: 4.0745x; 4.0745x over previous
import dataclasses
import functools

import jax
import jax.numpy as jnp
from jax import lax
from jax.experimental import pallas as pl
from jax.experimental.pallas import tpu as pltpu
from jax.experimental.pallas import tpu_sc as plsc

N = 10000
C = 128
K = 25
KP = 32

NW = 32
AD = 4
RG = 4
GD = 2
NG = 39
ROWS_PER = 320
NGRP = ROWS_PER // RG
NP = NW * ROWS_PER
XPAD_ROWS = N + 40


def _sage_body(a_hbm, xpad_hbm, out_hbm, abuf0, abuf1, abuf2, abuf3,
               idxb, gbuf, aggb, clist, sa0, sa1, sa2, sa3, sg0, sg1):
    wid = lax.axis_index("s") * 2 + lax.axis_index("c")
    base = wid * ROWS_PER
    iota16 = lax.iota(jnp.int32, 16)
    zfill_lo = N + iota16
    zfill_hi = N + 16 + iota16
    zeros16 = jnp.zeros((16,), jnp.int32)
    sa = (sa0, sa1, sa2, sa3)
    sg = (sg0, sg1)
    abufs = (abuf0, abuf1, abuf2, abuf3)
    lane_eq = [iota16 == t for t in range(16)]

    def a_row(r):
        return jnp.minimum(base + r, N - 1)

    def start_a(r, p):
        pltpu.make_async_copy(a_hbm.at[pl.ds(a_row(r) * N, N)], abufs[p],
                              sa[p]).start()

    def wait_a(p):
        pltpu.make_async_copy(a_hbm.at[pl.ds(0, N)], abufs[p],
                              sa[p]).wait()

    def start_g(pg):
        pltpu.make_async_copy(xpad_hbm.at[idxb.at[pg]], gbuf.at[pg],
                              sg[pg]).start()

    def wait_g(pg):
        pltpu.make_async_copy(xpad_hbm.at[idxb.at[pg]], gbuf.at[pg],
                              sg[pg]).wait()

    def scan_row(pa, pg, rg):
        idxb[pg, pl.ds(rg * KP, 16)] = zfill_lo
        idxb[pg, pl.ds(rg * KP + 16, 16)] = zfill_hi

        def compact_chunks(flags, g, gcnt):
            m2 = flags != 0
            pos = gcnt + plsc.cumsum(m2.astype(jnp.int32)) - 1
            sm = jnp.logical_and(m2, pos < KP)
            posc = jnp.minimum(pos, KP - 1)
            plsc.store_scatter(clist, [posc], iota16 + g * 16, mask=sm)
            return gcnt + plsc.all_reduce_population_count(m2)

        def group(g, gcnt):
            flags = zeros16
            for t in range(16):
                v = abufs[pa][pl.ds(g * 256 + t * 16, 16)]
                nz = lax.shift_left(plsc.bitcast(v, jnp.int32), 1) != 0
                pc = plsc.all_reduce_population_count(nz)
                flags = jnp.where(lane_eq[t], pc, flags)
            return compact_chunks(flags, g, gcnt)

        gcnt = lax.fori_loop(0, NG, group, zeros16, unroll=False)
        vlast = abufs[pa][pl.ds(NG * 256, 16)]
        nzl = lax.shift_left(plsc.bitcast(vlast, jnp.int32), 1) != 0
        pcl = plsc.all_reduce_population_count(nzl)
        gcnt = compact_chunks(jnp.where(lane_eq[0], pcl, zeros16), NG, gcnt)
        nchunks = jnp.minimum(jnp.max(gcnt), KP)

        def chunk(j, cnt):
            jv = jnp.full((16,), 0, jnp.int32) + j
            cid = plsc.load_gather(clist, [jv])
            cols = cid * 16 + iota16
            v = plsc.load_gather(abufs[pa], [cols])
            m = lax.shift_left(plsc.bitcast(v, jnp.int32), 1) != 0
            pos = cnt + plsc.cumsum(m.astype(jnp.int32)) - 1
            sm = jnp.logical_and(m, pos < K)
            posc = jnp.minimum(pos, KP - 1) + rg * KP
            plsc.store_scatter(idxb.at[pg], [posc], cols, mask=sm)
            return cnt + plsc.all_reduce_population_count(m)

        lax.fori_loop(0, nchunks, chunk, zeros16, unroll=False)

    def max_row(pg, rg, rmax):
        def mstep(k, accs):
            return tuple(
                jnp.maximum(a, gbuf[pg, rg * KP + k, pl.ds(cch * 16, 16)])
                for cch, a in enumerate(accs))
        accs = tuple(gbuf[pg, rg * KP, pl.ds(cch * 16, 16)]
                     for cch in range(C // 16))
        accs = lax.fori_loop(1, K, mstep, accs, unroll=4)
        for cch in range(C // 16):
            aggb[rmax, pl.ds(cch * 16, 16)] = accs[cch]

    for u in range(AD):
        start_a(u, u)

    @pl.loop(0, NGRP + GD, step=GD)
    def _(g0):
        for ug in range(GD):
            grp = g0 + ug

            @pl.when(grp < NGRP)
            def _():
                for rg in range(RG):
                    row = grp * RG + rg
                    wait_a(rg)
                    scan_row(rg, ug, rg)

                    @pl.when(row + AD < ROWS_PER)
                    def _():
                        start_a(row + AD, rg)
                start_g(ug)

            gmax = grp - 1

            @pl.when(jnp.logical_and(gmax >= 0, gmax < NGRP))
            def _():
                wait_g(1 - ug)
                for rg in range(RG):
                    max_row(1 - ug, rg, gmax * RG + rg)

    pltpu.sync_copy(aggb, out_hbm.at[pl.ds(base, ROWS_PER)])


def _sage_sc(A1, xpad):
    mesh = plsc.VectorSubcoreMesh(core_axis_name="c", subcore_axis_name="s")
    cp = pltpu.CompilerParams()
    if "needs_layout_passes" in pltpu.CompilerParams.__dataclass_fields__:
        cp = dataclasses.replace(cp, needs_layout_passes=False)
    kfn = functools.partial(
        pl.kernel,
        mesh=mesh,
        compiler_params=cp,
        out_type=jax.ShapeDtypeStruct((NP, C), jnp.float32),
        scratch_types=(
            [pltpu.VMEM((N,), jnp.float32)] * AD
            + [pltpu.VMEM((GD, RG * KP), jnp.int32),
               pltpu.VMEM((GD, RG * KP, C), jnp.float32),
               pltpu.VMEM((ROWS_PER, C), jnp.float32),
               pltpu.VMEM((KP,), jnp.int32)]
            + [pltpu.SemaphoreType.DMA] * (AD + GD)
        ),
    )(_sage_body)
    return kfn(A1, xpad)


def _mlp_body(x_ref, a_ref, w1_ref, w2_ref, b_ref, o_ref):
    acc = jnp.dot(x_ref[...], w1_ref[...], preferred_element_type=jnp.float32)
    acc += jnp.dot(a_ref[...], w2_ref[...], preferred_element_type=jnp.float32)
    o_ref[...] = jnp.maximum(acc + b_ref[...], 0.0)


def _mlp(X2, agg, W, b):
    MB = 1000
    return pl.pallas_call(
        _mlp_body,
        grid=(N // MB,),
        in_specs=[
            pl.BlockSpec((MB, C), lambda i: (i, 0)),
            pl.BlockSpec((MB, C), lambda i: (i, 0)),
            pl.BlockSpec((C, C), lambda i: (0, 0)),
            pl.BlockSpec((C, C), lambda i: (0, 0)),
            pl.BlockSpec((1, C), lambda i: (0, 0)),
        ],
        out_specs=pl.BlockSpec((MB, C), lambda i: (i, 0)),
        out_shape=jax.ShapeDtypeStruct((N, C), jnp.float32),
    )(X2, agg, W[:C], W[C:], b[None])


def kernel(A, X, agg_weights, agg_bias):
    X2 = X[0]
    A1 = jnp.reshape(A, (N * N,))
    xpad = jnp.pad(X2, ((0, XPAD_ROWS - N), (0, 0)))
    agg = _sage_sc(A1, xpad)[:N]
    out = _mlp(X2, agg, agg_weights, agg_bias)
    return out[None]


# speedup vs baseline: 6.5635x; 1.6109x over previous
import dataclasses
import functools

import jax
import jax.numpy as jnp
from jax import lax
from jax.experimental import pallas as pl
from jax.experimental.pallas import tpu as pltpu
from jax.experimental.pallas import tpu_sc as plsc

N = 10000
C = 128
K = 25
KP = 32

NW = 32
AD = 4
RG = 4
GD = 2
NG = 39
ROWS_PER = 320
NGRP = ROWS_PER // RG
NP = NW * ROWS_PER
XPAD_ROWS = N + 40


def _sage_body(a_hbm, xpad_hbm, out_hbm, abuf0, abuf1, abuf2, abuf3,
               idxb, gbuf, aggb, clist, sa0, sa1, sa2, sa3, sg0, sg1):
    wid = lax.axis_index("s") * 2 + lax.axis_index("c")
    base = wid * ROWS_PER
    iota16 = lax.iota(jnp.int32, 16)
    zfill_lo = N + iota16
    zfill_hi = N + 16 + iota16
    zeros16 = jnp.zeros((16,), jnp.int32)
    sa = (sa0, sa1, sa2, sa3)
    sg = (sg0, sg1)
    abufs = (abuf0, abuf1, abuf2, abuf3)
    lane_eq = [iota16 == t for t in range(16)]

    def a_row(r):
        return jnp.minimum(base + r, N - 1)

    def start_a(r, p):
        pltpu.make_async_copy(a_hbm.at[a_row(r)], abufs[p],
                              sa[p]).start()

    def wait_a(p):
        pltpu.make_async_copy(a_hbm.at[0], abufs[p],
                              sa[p]).wait()

    def start_g(pg):
        pltpu.make_async_copy(xpad_hbm.at[idxb.at[pg]], gbuf.at[pg],
                              sg[pg]).start()

    def wait_g(pg):
        pltpu.make_async_copy(xpad_hbm.at[idxb.at[pg]], gbuf.at[pg],
                              sg[pg]).wait()

    def scan_row(pa, pg, rg):
        idxb[pg, pl.ds(rg * KP, 16)] = zfill_lo
        idxb[pg, pl.ds(rg * KP + 16, 16)] = zfill_hi

        def compact_chunks(flags, g, gcnt):
            m2 = flags != 0
            pos = gcnt + plsc.cumsum(m2.astype(jnp.int32)) - 1
            sm = jnp.logical_and(m2, pos < KP)
            posc = jnp.minimum(pos, KP - 1)
            plsc.store_scatter(clist, [posc], iota16 + g * 16, mask=sm)
            return gcnt + plsc.all_reduce_population_count(m2)

        def group(g, gcnt):
            flags = zeros16
            for t in range(16):
                v = abufs[pa][pl.ds(g * 256 + t * 16, 16)]
                nz = lax.shift_left(plsc.bitcast(v, jnp.int32), 1) != 0
                pc = plsc.all_reduce_population_count(nz)
                flags = jnp.where(lane_eq[t], pc, flags)
            return compact_chunks(flags, g, gcnt)

        gcnt = lax.fori_loop(0, NG, group, zeros16, unroll=False)
        vlast = abufs[pa][pl.ds(NG * 256, 16)]
        nzl = lax.shift_left(plsc.bitcast(vlast, jnp.int32), 1) != 0
        pcl = plsc.all_reduce_population_count(nzl)
        gcnt = compact_chunks(jnp.where(lane_eq[0], pcl, zeros16), NG, gcnt)
        nchunks = jnp.minimum(jnp.max(gcnt), KP)

        def chunk(j, cnt):
            jv = jnp.full((16,), 0, jnp.int32) + j
            cid = plsc.load_gather(clist, [jv])
            cols = cid * 16 + iota16
            v = plsc.load_gather(abufs[pa], [cols])
            m = lax.shift_left(plsc.bitcast(v, jnp.int32), 1) != 0
            pos = cnt + plsc.cumsum(m.astype(jnp.int32)) - 1
            sm = jnp.logical_and(m, pos < K)
            posc = jnp.minimum(pos, KP - 1) + rg * KP
            plsc.store_scatter(idxb.at[pg], [posc], cols, mask=sm)
            return cnt + plsc.all_reduce_population_count(m)

        lax.fori_loop(0, nchunks, chunk, zeros16, unroll=False)

    def max_row(pg, rg, rmax):
        def mstep(k, accs):
            return tuple(
                jnp.maximum(a, gbuf[pg, rg * KP + k, pl.ds(cch * 16, 16)])
                for cch, a in enumerate(accs))
        accs = tuple(gbuf[pg, rg * KP, pl.ds(cch * 16, 16)]
                     for cch in range(C // 16))
        accs = lax.fori_loop(1, K, mstep, accs, unroll=4)
        for cch in range(C // 16):
            aggb[rmax, pl.ds(cch * 16, 16)] = accs[cch]

    for u in range(AD):
        start_a(u, u)

    @pl.loop(0, NGRP + GD, step=GD)
    def _(g0):
        for ug in range(GD):
            grp = g0 + ug

            @pl.when(grp < NGRP)
            def _():
                for rg in range(RG):
                    row = grp * RG + rg
                    wait_a(rg)
                    scan_row(rg, ug, rg)

                    @pl.when(row + AD < ROWS_PER)
                    def _():
                        start_a(row + AD, rg)
                start_g(ug)

            gmax = grp - 1

            @pl.when(jnp.logical_and(gmax >= 0, gmax < NGRP))
            def _():
                wait_g(1 - ug)
                for rg in range(RG):
                    max_row(1 - ug, rg, gmax * RG + rg)

    pltpu.sync_copy(aggb, out_hbm.at[pl.ds(base, ROWS_PER)])


def _sage_sc(A2, xpad):
    mesh = plsc.VectorSubcoreMesh(core_axis_name="c", subcore_axis_name="s")
    cp = pltpu.CompilerParams()
    if "needs_layout_passes" in pltpu.CompilerParams.__dataclass_fields__:
        cp = dataclasses.replace(cp, needs_layout_passes=False)
    kfn = functools.partial(
        pl.kernel,
        mesh=mesh,
        compiler_params=cp,
        out_type=jax.ShapeDtypeStruct((NP, C), jnp.float32),
        scratch_types=(
            [pltpu.VMEM((N,), jnp.float32)] * AD
            + [pltpu.VMEM((GD, RG * KP), jnp.int32),
               pltpu.VMEM((GD, RG * KP, C), jnp.float32),
               pltpu.VMEM((ROWS_PER, C), jnp.float32),
               pltpu.VMEM((KP,), jnp.int32)]
            + [pltpu.SemaphoreType.DMA] * (AD + GD)
        ),
    )(_sage_body)
    return kfn(A2, xpad)


def _mlp_body(x_ref, a_ref, w1_ref, w2_ref, b_ref, o_ref):
    acc = jnp.dot(x_ref[...], w1_ref[...], preferred_element_type=jnp.float32)
    acc += jnp.dot(a_ref[...], w2_ref[...], preferred_element_type=jnp.float32)
    o_ref[...] = jnp.maximum(acc + b_ref[...], 0.0)


def _mlp(X2, agg, W, b):
    MB = 1000
    return pl.pallas_call(
        _mlp_body,
        grid=(N // MB,),
        in_specs=[
            pl.BlockSpec((MB, C), lambda i: (i, 0)),
            pl.BlockSpec((MB, C), lambda i: (i, 0)),
            pl.BlockSpec((C, C), lambda i: (0, 0)),
            pl.BlockSpec((C, C), lambda i: (0, 0)),
            pl.BlockSpec((1, C), lambda i: (0, 0)),
        ],
        out_specs=pl.BlockSpec((MB, C), lambda i: (i, 0)),
        out_shape=jax.ShapeDtypeStruct((N, C), jnp.float32),
    )(X2, agg, W[:C], W[C:], b[None])


def kernel(A, X, agg_weights, agg_bias):
    X2 = X[0]
    A2 = A[0]
    xpad = jnp.pad(X2, ((0, XPAD_ROWS - N), (0, 0)))
    agg = _sage_sc(A2, xpad)[:N]
    out = _mlp(X2, agg, agg_weights, agg_bias)
    return out[None]


# speedup vs baseline: 6.6205x; 1.0087x over previous
import dataclasses
import functools

import jax
import jax.numpy as jnp
from jax import lax
from jax.experimental import pallas as pl
from jax.experimental.pallas import tpu as pltpu
from jax.experimental.pallas import tpu_sc as plsc

N = 10000
C = 128
K = 25
KP = 32

NW = 32
AD = 4
RG = 4
GD = 2
NG = 39
ROWS_PER = 320
NGRP = ROWS_PER // RG
NP = NW * ROWS_PER
XPAD_ROWS = N + 40


def _sage_body(a_hbm, xpad_hbm, out_hbm, abuf0, abuf1, abuf2, abuf3,
               idxb, gbuf, aggb, clist, sa0, sa1, sa2, sa3, sg0, sg1):
    wid = lax.axis_index("s") * 2 + lax.axis_index("c")
    base = wid * ROWS_PER
    iota16 = lax.iota(jnp.int32, 16)
    zfill_lo = N + iota16
    zfill_hi = N + 16 + iota16
    zeros16 = jnp.zeros((16,), jnp.int32)
    sa = (sa0, sa1, sa2, sa3)
    sg = (sg0, sg1)
    abufs = (abuf0, abuf1, abuf2, abuf3)
    lane_eq = [iota16 == t for t in range(16)]

    def a_row(r):
        return jnp.minimum(base + r, N - 1)

    def start_a(r, p):
        pltpu.make_async_copy(a_hbm.at[a_row(r)], abufs[p],
                              sa[p]).start()

    def wait_a(p):
        pltpu.make_async_copy(a_hbm.at[0], abufs[p],
                              sa[p]).wait()

    def start_g(pg):
        pltpu.make_async_copy(xpad_hbm.at[idxb.at[pg]], gbuf.at[pg],
                              sg[pg]).start()

    def wait_g(pg):
        pltpu.make_async_copy(xpad_hbm.at[idxb.at[pg]], gbuf.at[pg],
                              sg[pg]).wait()

    def scan_row(pa, pg, rg):
        idxb[pg, pl.ds(rg * KP, 16)] = zfill_lo
        idxb[pg, pl.ds(rg * KP + 16, 16)] = zfill_hi

        def compact_chunks(flags, g, gcnt):
            m2 = flags != 0
            pos = gcnt + plsc.cumsum(m2.astype(jnp.int32)) - 1
            sm = jnp.logical_and(m2, pos < KP)
            posc = jnp.minimum(pos, KP - 1)
            plsc.store_scatter(clist, [posc], iota16 + g * 16, mask=sm)
            return gcnt + plsc.all_reduce_population_count(m2)

        def group(g, gcnt):
            flags = zeros16
            for t in range(16):
                v = abufs[pa][pl.ds(g * 256 + t * 16, 16)]
                nz = lax.shift_left(plsc.bitcast(v, jnp.int32), 1) != 0
                pc = plsc.all_reduce_population_count(nz)
                flags = jnp.where(lane_eq[t], pc, flags)
            return compact_chunks(flags, g, gcnt)

        gcnt = lax.fori_loop(0, NG, group, zeros16, unroll=False)
        vlast = abufs[pa][pl.ds(NG * 256, 16)]
        nzl = lax.shift_left(plsc.bitcast(vlast, jnp.int32), 1) != 0
        pcl = plsc.all_reduce_population_count(nzl)
        gcnt = compact_chunks(jnp.where(lane_eq[0], pcl, zeros16), NG, gcnt)
        nchunks = jnp.minimum(jnp.max(gcnt), KP)

        def chunk(j, cnt):
            jv = jnp.full((16,), 0, jnp.int32) + j
            cid = plsc.load_gather(clist, [jv])
            cols = cid * 16 + iota16
            v = plsc.load_gather(abufs[pa], [cols])
            m = lax.shift_left(plsc.bitcast(v, jnp.int32), 1) != 0
            pos = cnt + plsc.cumsum(m.astype(jnp.int32)) - 1
            sm = jnp.logical_and(m, pos < K)
            posc = jnp.minimum(pos, KP - 1) + rg * KP
            plsc.store_scatter(idxb.at[pg], [posc], cols, mask=sm)
            return cnt + plsc.all_reduce_population_count(m)

        lax.fori_loop(0, nchunks, chunk, zeros16, unroll=False)

    def max_row(pg, rg, rmax):
        def mstep(k, accs):
            return tuple(
                jnp.maximum(a, gbuf[pg, rg * KP + k, pl.ds(cch * 16, 16)])
                for cch, a in enumerate(accs))
        accs = tuple(gbuf[pg, rg * KP, pl.ds(cch * 16, 16)]
                     for cch in range(C // 16))
        accs = lax.fori_loop(1, K, mstep, accs, unroll=4)
        for cch in range(C // 16):
            aggb[rmax, pl.ds(cch * 16, 16)] = accs[cch]

    for u in range(AD):
        start_a(u, u)

    @pl.loop(0, NGRP + GD, step=GD)
    def _(g0):
        for ug in range(GD):
            grp = g0 + ug

            @pl.when(grp < NGRP)
            def _():
                for rg in range(RG):
                    row = grp * RG + rg
                    wait_a(rg)
                    scan_row(rg, ug, rg)

                    @pl.when(row + AD < ROWS_PER)
                    def _():
                        start_a(row + AD, rg)
                start_g(ug)

            gmax = grp - 1

            @pl.when(jnp.logical_and(gmax >= 0, gmax < NGRP))
            def _():
                wait_g(1 - ug)
                for rg in range(RG):
                    max_row(1 - ug, rg, gmax * RG + rg)

    pltpu.sync_copy(aggb, out_hbm.at[pl.ds(base, ROWS_PER)])


def _sage_sc(A2, xpad):
    mesh = plsc.VectorSubcoreMesh(core_axis_name="c", subcore_axis_name="s")
    cp = pltpu.CompilerParams()
    if "needs_layout_passes" in pltpu.CompilerParams.__dataclass_fields__:
        cp = dataclasses.replace(cp, needs_layout_passes=False)
    kfn = functools.partial(
        pl.kernel,
        mesh=mesh,
        compiler_params=cp,
        out_type=jax.ShapeDtypeStruct((NP, C), jnp.float32),
        scratch_types=(
            [pltpu.VMEM((N,), jnp.float32)] * AD
            + [pltpu.VMEM((GD, RG * KP), jnp.int32),
               pltpu.VMEM((GD, RG * KP, C), jnp.float32),
               pltpu.VMEM((ROWS_PER, C), jnp.float32),
               pltpu.VMEM((KP,), jnp.int32)]
            + [pltpu.SemaphoreType.DMA] * (AD + GD)
        ),
    )(_sage_body)
    return kfn(A2, xpad)


def _mlp_body(x_ref, a_ref, w1_ref, w2_ref, b_ref, o_ref):
    acc = jnp.dot(x_ref[...], w1_ref[...], preferred_element_type=jnp.float32)
    acc += jnp.dot(a_ref[...], w2_ref[...], preferred_element_type=jnp.float32)
    o_ref[...] = jnp.maximum(acc + b_ref[...], 0.0)


def _mlp(X2, agg, W, b):
    MB = 1000
    return pl.pallas_call(
        _mlp_body,
        grid=(N // MB,),
        in_specs=[
            pl.BlockSpec((MB, C), lambda i: (i, 0)),
            pl.BlockSpec((MB, C), lambda i: (i, 0)),
            pl.BlockSpec((C, C), lambda i: (0, 0)),
            pl.BlockSpec((C, C), lambda i: (0, 0)),
            pl.BlockSpec((1, C), lambda i: (0, 0)),
        ],
        out_specs=pl.BlockSpec((MB, C), lambda i: (i, 0)),
        out_shape=jax.ShapeDtypeStruct((N, C), jnp.float32),
    )(X2, agg, W[:C], W[C:], b[None])


def kernel(A, X, agg_weights, agg_bias):
    X2 = X[0]
    A2 = A[0]
    xpad = jnp.pad(X2, ((0, XPAD_ROWS - N), (0, 0)))
    agg = _sage_sc(A2, xpad)
    out = _mlp(X2, agg, agg_weights, agg_bias)
    return out[None]
